# baseline (speedup 1.0000x reference)
import functools

import jax
import jax.numpy as jnp
from jax import lax
from jax.experimental import pallas as pl
from jax.experimental.pallas import tpu as pltpu
from jax.experimental.pallas import tpu_sc as plsc

B, C, T, K = 32, 256, 4096, 64
BETA = 10.0

NC = 2
NS = 16
NW = NC * NS
ROWS = B * K
RPW = ROWS // NW
CHUNK = 8
NCHUNK = RPW // CHUNK


def _alpha_body(a_ref, p_ref, w_ref, g_ref):
    a = a_ref[...]
    s = a * (1.0 / BETA)
    m = jnp.max(s, axis=1, keepdims=True)
    e = jnp.exp(s - m)
    p = e / jnp.sum(e, axis=1, keepdims=True)

    am = jnp.max(a, axis=1, keepdims=True)
    iota_c = lax.broadcasted_iota(jnp.int32, (K, C), 1)
    idx = jnp.min(jnp.where(a == am, iota_c, C), axis=1)
    hard = (iota_c == idx[:, None]).astype(a.dtype)

    p_ref[...] = p
    w_ref[...] = p + (hard - p)
    bi = lax.broadcasted_iota(jnp.int32, (B, K), 0)
    g_ref[...] = bi * C + idx[None, :]


_alpha_call = pl.pallas_call(
    _alpha_body,
    out_shape=(
        jax.ShapeDtypeStruct((K, C), jnp.float32),
        jax.ShapeDtypeStruct((K, C), jnp.float32),
        jax.ShapeDtypeStruct((B, K), jnp.int32),
    ),
)

_mesh = plsc.VectorSubcoreMesh(core_axis_name="c", subcore_axis_name="s")

@functools.partial(
    pl.kernel,
    mesh=_mesh,
    out_type=jax.ShapeDtypeStruct((ROWS, T), jnp.float32),
    scratch_types=[
        pltpu.VMEM((RPW,), jnp.int32),
        pltpu.VMEM((CHUNK, T), jnp.float32),
        pltpu.VMEM((CHUNK, T), jnp.float32),
        pltpu.SemaphoreType.DMA,
        pltpu.SemaphoreType.DMA,
        pltpu.SemaphoreType.DMA,
        pltpu.SemaphoreType.DMA,
    ],
)
def _gather(x_hbm, g_hbm, z_hbm, idx_v, buf0, buf1, gs0, gs1, ss0, ss1):
    wid = lax.axis_index("s") * NC + lax.axis_index("c")
    base = wid * RPW
    pltpu.sync_copy(g_hbm.at[wid], idx_v)

    def gath(c, buf, sem):
        return pltpu.async_copy(
            x_hbm.at[idx_v.at[pl.ds(c * CHUNK, CHUNK)]], buf, sem)

    def scat(c, buf, sem):
        return pltpu.async_copy(
            buf, z_hbm.at[pl.ds(base + c * CHUNK, CHUNK)], sem)

    def gath_wait(c, buf, sem):
        pltpu.make_async_copy(
            x_hbm.at[idx_v.at[pl.ds(c * CHUNK, CHUNK)]], buf, sem).wait()

    def scat_wait(c, buf, sem):
        pltpu.make_async_copy(
            buf, z_hbm.at[pl.ds(base + c * CHUNK, CHUNK)], sem).wait()

    gath(0, buf0, gs0)
    gath(1, buf1, gs1)

    def body(c0, _):
        c1 = c0 + 1
        gath_wait(c0, buf0, gs0)
        scat(c0, buf0, ss0)
        gath_wait(c1, buf1, gs1)
        scat(c1, buf1, ss1)

        @pl.when(c0 + 2 < NCHUNK)
        def _():
            scat_wait(c0, buf0, ss0)
            gath(c0 + 2, buf0, gs0)
            scat_wait(c1, buf1, ss1)
            gath(c1 + 2, buf1, gs1)

        return ()

    lax.fori_loop(0, NCHUNK // 2, lambda i, s: body(i * 2, s), ())
    scat_wait(NCHUNK - 2, buf0, ss0)
    scat_wait(NCHUNK - 1, buf1, ss1)


def kernel(x, alpha):
    p, w, g = _alpha_call(alpha)
    x2 = x.reshape(B * C, T)
    z = _gather(x2, g)
    return (z.reshape(B, 1, K, T), w, p)


# speedup vs baseline: 1.0548x; 1.0548x over previous
import functools

import jax
import jax.numpy as jnp
from jax import lax
from jax.experimental import pallas as pl
from jax.experimental.pallas import tpu as pltpu
from jax.experimental.pallas import tpu_sc as plsc

B, C, T, K = 32, 256, 4096, 64
BETA = 10.0

NC = 2
NS = 16
NW = NC * NS
ROWS = B * K
RPW = ROWS // NW
CHUNK = 4
NCHUNK = RPW // CHUNK


def _alpha_body(a_ref, p_ref, w_ref, g_ref):
    a = a_ref[...]
    s = a * (1.0 / BETA)
    m = jnp.max(s, axis=1, keepdims=True)
    e = jnp.exp(s - m)
    p = e / jnp.sum(e, axis=1, keepdims=True)

    am = jnp.max(a, axis=1, keepdims=True)
    iota_c = lax.broadcasted_iota(jnp.int32, (K, C), 1)
    idx = jnp.min(jnp.where(a == am, iota_c, C), axis=1)
    hard = (iota_c == idx[:, None]).astype(a.dtype)

    p_ref[...] = p
    w_ref[...] = p + (hard - p)
    bi = lax.broadcasted_iota(jnp.int32, (B, K), 0)
    g_ref[...] = bi * C + idx[None, :]


_alpha_call = pl.pallas_call(
    _alpha_body,
    out_shape=(
        jax.ShapeDtypeStruct((K, C), jnp.float32),
        jax.ShapeDtypeStruct((K, C), jnp.float32),
        jax.ShapeDtypeStruct((B, K), jnp.int32),
    ),
)

_mesh = plsc.VectorSubcoreMesh(core_axis_name="c", subcore_axis_name="s")

NBUF = 7


@functools.partial(
    pl.kernel,
    mesh=_mesh,
    out_type=jax.ShapeDtypeStruct((ROWS, T), jnp.float32),
    scratch_types=(
        [pltpu.VMEM((NCHUNK, CHUNK), jnp.int32)]
        + [pltpu.VMEM((CHUNK, T), jnp.float32)] * NBUF
        + [pltpu.SemaphoreType.DMA] * (2 * NBUF)
    ),
)
def _gather(x_hbm, g_hbm, z_hbm, idx_v, *bufs_sems):
    bufs = bufs_sems[:NBUF]
    gsems = bufs_sems[NBUF:2 * NBUF]
    ssems = bufs_sems[2 * NBUF:]
    wid = lax.axis_index("s") * NC + lax.axis_index("c")
    base = wid * RPW
    pltpu.sync_copy(g_hbm.at[wid], idx_v)

    gcp, scp = {}, {}
    for c in range(NBUF):
        gcp[c] = pltpu.async_copy(
            x_hbm.at[idx_v.at[c]], bufs[c % NBUF], gsems[c % NBUF])
    for c in range(NCHUNK):
        gcp[c].wait()
        scp[c] = pltpu.async_copy(
            bufs[c % NBUF],
            z_hbm.at[pl.ds(base + c * CHUNK, CHUNK)], ssems[c % NBUF])
        if c + NBUF < NCHUNK:
            scp[c].wait()
            gcp[c + NBUF] = pltpu.async_copy(
                x_hbm.at[idx_v.at[c + NBUF]],
                bufs[c % NBUF], gsems[c % NBUF])
    for c in range(max(0, NCHUNK - NBUF), NCHUNK):
        scp[c].wait()


def kernel(x, alpha):
    p, w, g = _alpha_call(alpha)
    x2 = x.reshape(B * C, T)
    z = _gather(x2, g.reshape(NW, NCHUNK, CHUNK))
    return (z.reshape(B, 1, K, T), w, p)


# speedup vs baseline: 1.0642x; 1.0090x over previous
import functools

import jax
import jax.numpy as jnp
from jax import lax
from jax.experimental import pallas as pl
from jax.experimental.pallas import tpu as pltpu
from jax.experimental.pallas import tpu_sc as plsc

B, C, T, K = 32, 256, 4096, 64
BETA = 10.0

NC = 2
NS = 16
NW = NC * NS
ROWS = B * K
RPW = ROWS // NW
CHUNK = 8
NCHUNK = RPW // CHUNK


def _alpha_body(a_ref, p_ref, w_ref, g_ref):
    a = a_ref[...]
    s = a * (1.0 / BETA)
    m = jnp.max(s, axis=1, keepdims=True)
    e = jnp.exp(s - m)
    p = e / jnp.sum(e, axis=1, keepdims=True)

    am = jnp.max(a, axis=1, keepdims=True)
    iota_c = lax.broadcasted_iota(jnp.int32, (K, C), 1)
    idx = jnp.min(jnp.where(a == am, iota_c, C), axis=1)
    hard = (iota_c == idx[:, None]).astype(a.dtype)

    p_ref[...] = p
    w_ref[...] = p + (hard - p)
    bi = lax.broadcasted_iota(jnp.int32, (B, K), 0)
    g_ref[...] = bi * C + idx[None, :]


_alpha_call = pl.pallas_call(
    _alpha_body,
    out_shape=(
        jax.ShapeDtypeStruct((K, C), jnp.float32),
        jax.ShapeDtypeStruct((K, C), jnp.float32),
        jax.ShapeDtypeStruct((B, K), jnp.int32),
    ),
)

_mesh = plsc.VectorSubcoreMesh(core_axis_name="c", subcore_axis_name="s")

NBUF = 3


@functools.partial(
    pl.kernel,
    mesh=_mesh,
    out_type=jax.ShapeDtypeStruct((ROWS, T), jnp.float32),
    scratch_types=(
        [pltpu.VMEM((NCHUNK, CHUNK), jnp.int32)]
        + [pltpu.VMEM((CHUNK, T), jnp.float32)] * NBUF
        + [pltpu.SemaphoreType.DMA] * (2 * NBUF)
    ),
)
def _gather(x_hbm, g_hbm, z_hbm, idx_v, *bufs_sems):
    bufs = bufs_sems[:NBUF]
    gsems = bufs_sems[NBUF:2 * NBUF]
    ssems = bufs_sems[2 * NBUF:]
    wid = lax.axis_index("s") * NC + lax.axis_index("c")
    base = wid * RPW
    pltpu.sync_copy(g_hbm.at[wid], idx_v)

    gcp, scp = {}, {}
    for c in range(NBUF):
        gcp[c] = pltpu.async_copy(
            x_hbm.at[idx_v.at[c]], bufs[c % NBUF], gsems[c % NBUF])
    for c in range(NCHUNK):
        gcp[c].wait()
        scp[c] = pltpu.async_copy(
            bufs[c % NBUF],
            z_hbm.at[pl.ds(base + c * CHUNK, CHUNK)], ssems[c % NBUF])
        if c + NBUF < NCHUNK:
            scp[c].wait()
            gcp[c + NBUF] = pltpu.async_copy(
                x_hbm.at[idx_v.at[c + NBUF]],
                bufs[c % NBUF], gsems[c % NBUF])
    for c in range(max(0, NCHUNK - NBUF), NCHUNK):
        scp[c].wait()


def kernel(x, alpha):
    p, w, g = _alpha_call(alpha)
    x2 = x.reshape(B * C, T)
    z = _gather(x2, g.reshape(NW, NCHUNK, CHUNK))
    return (z.reshape(B, 1, K, T), w, p)


# speedup vs baseline: 1.0661x; 1.0018x over previous
import functools

import jax
import jax.numpy as jnp
from jax import lax
from jax.experimental import pallas as pl
from jax.experimental.pallas import tpu as pltpu
from jax.experimental.pallas import tpu_sc as plsc

B, C, T, K = 32, 256, 4096, 64
BETA = 10.0

NC = 2
NS = 16
NW = NC * NS
ROWS = B * K
RPW = ROWS // NW
CHUNK = 8
NCHUNK = RPW // CHUNK


def _alpha_body(a_ref, p_ref, w_ref, g_ref):
    a = a_ref[...]
    s = a * (1.0 / BETA)
    m = jnp.max(s, axis=1, keepdims=True)
    e = jnp.exp(s - m)
    p = e / jnp.sum(e, axis=1, keepdims=True)

    am = jnp.max(a, axis=1, keepdims=True)
    iota_c = lax.broadcasted_iota(jnp.int32, (K, C), 1)
    idx = jnp.min(jnp.where(a == am, iota_c, C), axis=1)
    hard = (iota_c == idx[:, None]).astype(a.dtype)

    p_ref[...] = p
    w_ref[...] = p + (hard - p)
    idx2 = jnp.concatenate([idx, idx])
    bi = (lax.broadcasted_iota(jnp.int32, (ROWS // 128, 128), 0) * 2
          + lax.broadcasted_iota(jnp.int32, (ROWS // 128, 128), 1) // K)
    g_ref[...] = bi * C + idx2[None, :]


_alpha_call = pl.pallas_call(
    _alpha_body,
    out_shape=(
        jax.ShapeDtypeStruct((K, C), jnp.float32),
        jax.ShapeDtypeStruct((K, C), jnp.float32),
        jax.ShapeDtypeStruct((ROWS // 128, 128), jnp.int32),
    ),
)

_mesh = plsc.VectorSubcoreMesh(core_axis_name="c", subcore_axis_name="s")

NBUF = 3


@functools.partial(
    pl.kernel,
    mesh=_mesh,
    out_type=jax.ShapeDtypeStruct((ROWS, T), jnp.float32),
    scratch_types=(
        [pltpu.VMEM((2 * RPW,), jnp.int32)]
        + [pltpu.VMEM((CHUNK, T), jnp.float32)] * NBUF
        + [pltpu.SemaphoreType.DMA] * (2 * NBUF)
    ),
)
def _gather(x_hbm, g_hbm, z_hbm, idx_v, *bufs_sems):
    bufs = bufs_sems[:NBUF]
    gsems = bufs_sems[NBUF:2 * NBUF]
    ssems = bufs_sems[2 * NBUF:]
    wid = lax.axis_index("s") * NC + lax.axis_index("c")
    base = wid * RPW
    pltpu.sync_copy(g_hbm.at[wid >> 1], idx_v)
    ioff = pl.multiple_of((wid & 1) * RPW, 8)

    gcp, scp = {}, {}
    for c in range(NBUF):
        gcp[c] = pltpu.async_copy(
            x_hbm.at[idx_v.at[pl.ds(ioff + c * CHUNK, CHUNK)]],
            bufs[c % NBUF], gsems[c % NBUF])
    for c in range(NCHUNK):
        gcp[c].wait()
        scp[c] = pltpu.async_copy(
            bufs[c % NBUF],
            z_hbm.at[pl.ds(base + c * CHUNK, CHUNK)], ssems[c % NBUF])
        if c + NBUF < NCHUNK:
            scp[c].wait()
            gcp[c + NBUF] = pltpu.async_copy(
                x_hbm.at[idx_v.at[pl.ds(ioff + (c + NBUF) * CHUNK, CHUNK)]],
                bufs[c % NBUF], gsems[c % NBUF])
    for c in range(max(0, NCHUNK - NBUF), NCHUNK):
        scp[c].wait()


def kernel(x, alpha):
    p, w, g = _alpha_call(alpha)
    x2 = x.reshape(B * C, T)
    z = _gather(x2, g)
    return (z.reshape(B, 1, K, T), w, p)


# speedup vs baseline: 1.0670x; 1.0009x over previous
import functools

import jax
import jax.numpy as jnp
from jax import lax
from jax.experimental import pallas as pl
from jax.experimental.pallas import tpu as pltpu
from jax.experimental.pallas import tpu_sc as plsc

B, C, T, K = 32, 256, 4096, 64
BETA = 10.0

NC = 2
NS = 16
NW = NC * NS
ROWS = B * K
RPW = ROWS // NW
CHUNK = 8
NCHUNK = RPW // CHUNK


def _argmax_body(a_ref, g_ref):
    a = a_ref[...]
    am = jnp.max(a, axis=1, keepdims=True)
    iota_c = lax.broadcasted_iota(jnp.int32, (K, C), 1)
    idx = jnp.min(jnp.where(a == am, iota_c, C), axis=1)
    idx2 = jnp.concatenate([idx, idx])
    bi = (lax.broadcasted_iota(jnp.int32, (ROWS // 128, 128), 0) * 2
          + lax.broadcasted_iota(jnp.int32, (ROWS // 128, 128), 1) // K)
    g_ref[...] = bi * C + idx2[None, :]


def _softmax_body(a_ref, p_ref, w_ref):
    a = a_ref[...]
    s = a * (1.0 / BETA)
    m = jnp.max(s, axis=1, keepdims=True)
    e = jnp.exp(s - m)
    p = e / jnp.sum(e, axis=1, keepdims=True)

    am = jnp.max(a, axis=1, keepdims=True)
    iota_c = lax.broadcasted_iota(jnp.int32, (K, C), 1)
    idx = jnp.min(jnp.where(a == am, iota_c, C), axis=1)
    hard = (iota_c == idx[:, None]).astype(a.dtype)

    p_ref[...] = p
    w_ref[...] = p + (hard - p)


_argmax_call = pl.pallas_call(
    _argmax_body,
    out_shape=jax.ShapeDtypeStruct((ROWS // 128, 128), jnp.int32),
)

_softmax_call = pl.pallas_call(
    _softmax_body,
    out_shape=(
        jax.ShapeDtypeStruct((K, C), jnp.float32),
        jax.ShapeDtypeStruct((K, C), jnp.float32),
    ),
)

_mesh = plsc.VectorSubcoreMesh(core_axis_name="c", subcore_axis_name="s")

NBUF = 3


@functools.partial(
    pl.kernel,
    mesh=_mesh,
    out_type=jax.ShapeDtypeStruct((ROWS, T), jnp.float32),
    scratch_types=(
        [pltpu.VMEM((2 * RPW,), jnp.int32)]
        + [pltpu.VMEM((CHUNK, T), jnp.float32)] * NBUF
        + [pltpu.SemaphoreType.DMA] * (2 * NBUF)
    ),
)
def _gather(x_hbm, g_hbm, z_hbm, idx_v, *bufs_sems):
    bufs = bufs_sems[:NBUF]
    gsems = bufs_sems[NBUF:2 * NBUF]
    ssems = bufs_sems[2 * NBUF:]
    wid = lax.axis_index("s") * NC + lax.axis_index("c")
    base = wid * RPW
    pltpu.sync_copy(g_hbm.at[wid >> 1], idx_v)
    ioff = pl.multiple_of((wid & 1) * RPW, 8)

    gcp, scp = {}, {}
    for c in range(NBUF):
        gcp[c] = pltpu.async_copy(
            x_hbm.at[idx_v.at[pl.ds(ioff + c * CHUNK, CHUNK)]],
            bufs[c % NBUF], gsems[c % NBUF])
    for c in range(NCHUNK):
        gcp[c].wait()
        scp[c] = pltpu.async_copy(
            bufs[c % NBUF],
            z_hbm.at[pl.ds(base + c * CHUNK, CHUNK)], ssems[c % NBUF])
        if c + NBUF < NCHUNK:
            scp[c].wait()
            gcp[c + NBUF] = pltpu.async_copy(
                x_hbm.at[idx_v.at[pl.ds(ioff + (c + NBUF) * CHUNK, CHUNK)]],
                bufs[c % NBUF], gsems[c % NBUF])
    for c in range(max(0, NCHUNK - NBUF), NCHUNK):
        scp[c].wait()


def kernel(x, alpha):
    g = _argmax_call(alpha)
    x2 = x.reshape(B * C, T)
    z = _gather(x2, g)
    p, w = _softmax_call(alpha)
    return (z.reshape(B, 1, K, T), w, p)
